# baseline (speedup 1.0000x reference)
import jax
import jax.numpy as jnp
from jax.experimental import pallas as pl
from jax.experimental.pallas import tpu as pltpu

B, T, D = 2, 2048, 1024
P = 8
K = 2
DFF = 1024
OFF_BIAS = 0.01
OFF_VAR = 0.01
NUDGE = 0.001

N = B * T
TB = 2048
NT = N // TB


def _moe_kernel(x_ref, keys_ref, bias_ref, w1_ref, w2_ref,
                out_ref, psum_ref, cnt_ref, sq_ref,
                xb_ref, cmb_ref):
    p = pl.program_id(1)

    @pl.when(p == 0)
    def _selector():
        x = x_ref[...]
        xb_ref[...] = x.astype(jnp.bfloat16)
        lt = jax.lax.dot_general(keys_ref[...], x, (((1,), (1,)), ((), ())),
                                 preferred_element_type=jnp.float32)
        lt = lt + bias_ref[...]
        m1 = jnp.max(lt, axis=0, keepdims=True)
        e = jnp.exp(lt - m1)
        probs = e / jnp.sum(e, axis=0, keepdims=True)
        iota = jax.lax.broadcasted_iota(jnp.int32, lt.shape, 0)
        arg1 = jnp.min(jnp.where(lt == m1, iota, P), axis=0, keepdims=True)
        masked = jnp.where(iota == arg1, -jnp.inf, lt)
        m2 = jnp.max(masked, axis=0, keepdims=True)
        arg2 = jnp.min(jnp.where(masked == m2, iota, P), axis=0, keepdims=True)
        w1v = 1.0 / (1.0 + jnp.exp(m2 - m1))
        sel1 = (iota == arg1).astype(jnp.float32)
        sel2 = (iota == arg2).astype(jnp.float32)
        cmb_ref[...] = (sel1 * w1v + sel2 * (1.0 - w1v)).T
        psum_ref[...] = jnp.sum(probs, axis=1).reshape(1, 1, P)
        cnt_ref[...] = jnp.sum(sel1 + sel2, axis=1).reshape(1, 1, P)
        sq_ref[...] = jnp.full((1, 1, P), jnp.sum(lt * lt), jnp.float32)

    h = jnp.dot(xb_ref[...], w1_ref[0].astype(jnp.bfloat16),
                preferred_element_type=jnp.float32)
    iota = jax.lax.broadcasted_iota(jnp.int32, (TB, P), 1)
    c = jnp.sum(cmb_ref[...] * (iota == p).astype(jnp.float32),
                axis=1, keepdims=True)
    hb = jnp.maximum(h.astype(jnp.bfloat16),
                     jnp.bfloat16(0.0)) * c.astype(jnp.bfloat16)
    y = jnp.dot(hb, w2_ref[0].astype(jnp.bfloat16),
                preferred_element_type=jnp.float32)

    @pl.when(p == 0)
    def _init():
        out_ref[...] = y

    @pl.when(p > 0)
    def _acc():
        out_ref[...] += y


@jax.jit
def kernel(tensor, biases, partitions, keys, W1, W2):
    del partitions
    x = tensor.reshape(N, D)
    bias2d = biases.reshape(P, 1)

    out, psum, cnt, sq = pl.pallas_call(
        _moe_kernel,
        grid=(NT, P),
        in_specs=[
            pl.BlockSpec((TB, D), lambda i, p: (i, 0)),
            pl.BlockSpec((P, D), lambda i, p: (0, 0)),
            pl.BlockSpec((P, 1), lambda i, p: (0, 0)),
            pl.BlockSpec((1, D, DFF), lambda i, p: (p, 0, 0)),
            pl.BlockSpec((1, DFF, D), lambda i, p: (p, 0, 0)),
        ],
        out_specs=[
            pl.BlockSpec((TB, D), lambda i, p: (i, 0)),
            pl.BlockSpec((1, 1, P), lambda i, p: (i, 0, 0)),
            pl.BlockSpec((1, 1, P), lambda i, p: (i, 0, 0)),
            pl.BlockSpec((1, 1, P), lambda i, p: (i, 0, 0)),
        ],
        out_shape=[
            jax.ShapeDtypeStruct((N, D), jnp.float32),
            jax.ShapeDtypeStruct((NT, 1, P), jnp.float32),
            jax.ShapeDtypeStruct((NT, 1, P), jnp.float32),
            jax.ShapeDtypeStruct((NT, 1, P), jnp.float32),
        ],
        scratch_shapes=[
            pltpu.VMEM((TB, D), jnp.bfloat16),
            pltpu.VMEM((TB, P), jnp.float32),
        ],
        compiler_params=pltpu.CompilerParams(
            vmem_limit_bytes=120 * 1024 * 1024,
            dimension_semantics=("parallel", "arbitrary")),
    )(x, keys, bias2d, W1, W2)

    mean_prob = jnp.sum(psum, axis=(0, 1)) / N
    load_frac = jnp.sum(cnt, axis=(0, 1)) / (N * K)
    off_bias_loss = OFF_BIAS * P * jnp.sum(mean_prob * load_frac)
    off_var_loss = OFF_VAR * jnp.var(load_frac)
    nudge_loss = NUDGE * jnp.sum(sq[:, 0, 0]) / (N * P)
    loss = off_bias_loss + off_var_loss + nudge_loss
    return out.reshape(B, T, D), loss


# speedup vs baseline: 1.0344x; 1.0344x over previous
import jax
import jax.numpy as jnp
from jax.experimental import pallas as pl
from jax.experimental.pallas import tpu as pltpu

B, T, D = 2, 2048, 1024
P = 8
K = 2
DFF = 1024
OFF_BIAS = 0.01
OFF_VAR = 0.01
NUDGE = 0.001

N = B * T
TB = 1024
NT = N // TB
EB = 2


def _moe_kernel(x_ref, keys_ref, bias_ref, w1_ref, w2_ref,
                out_ref, psum_ref, cnt_ref, sq_ref,
                xb_ref, cmb_ref, h2_ref):
    p = pl.program_id(1)

    @pl.when(p == 0)
    def _selector():
        x = x_ref[...]
        xb_ref[...] = x.astype(jnp.bfloat16)
        lt = jax.lax.dot_general(keys_ref[...], x, (((1,), (1,)), ((), ())),
                                 preferred_element_type=jnp.float32)
        lt = lt + bias_ref[...]
        m1 = jnp.max(lt, axis=0, keepdims=True)
        e = jnp.exp(lt - m1)
        probs = e / jnp.sum(e, axis=0, keepdims=True)
        iota = jax.lax.broadcasted_iota(jnp.int32, lt.shape, 0)
        arg1 = jnp.min(jnp.where(lt == m1, iota, P), axis=0, keepdims=True)
        masked = jnp.where(iota == arg1, -jnp.inf, lt)
        m2 = jnp.max(masked, axis=0, keepdims=True)
        arg2 = jnp.min(jnp.where(masked == m2, iota, P), axis=0, keepdims=True)
        w1v = 1.0 / (1.0 + jnp.exp(m2 - m1))
        sel1 = (iota == arg1).astype(jnp.float32)
        sel2 = (iota == arg2).astype(jnp.float32)
        cmb_ref[...] = (sel1 * w1v + sel2 * (1.0 - w1v)).T
        psum_ref[...] = jnp.sum(probs, axis=1).reshape(1, 1, P)
        cnt_ref[...] = jnp.sum(sel1 + sel2, axis=1).reshape(1, 1, P)
        sq_ref[...] = jnp.full((1, 1, P), jnp.sum(lt * lt), jnp.float32)

    iota = jax.lax.broadcasted_iota(jnp.int32, (TB, P), 1)
    for e in range(EB):
        pe = p * EB + e
        h = jnp.dot(xb_ref[...], w1_ref[e].astype(jnp.bfloat16),
                    preferred_element_type=jnp.float32)
        c = jnp.sum(cmb_ref[...] * (iota == pe).astype(jnp.float32),
                    axis=1, keepdims=True)
        h2_ref[:, e * DFF:(e + 1) * DFF] = jnp.maximum(
            h.astype(jnp.bfloat16), jnp.bfloat16(0.0)) * c.astype(jnp.bfloat16)
    y = jnp.dot(h2_ref[...], w2_ref[...].astype(jnp.bfloat16),
                preferred_element_type=jnp.float32)

    @pl.when(p == 0)
    def _init():
        out_ref[...] = y

    @pl.when(p > 0)
    def _acc():
        out_ref[...] += y


@jax.jit
def kernel(tensor, biases, partitions, keys, W1, W2):
    del partitions
    x = tensor.reshape(N, D)
    bias2d = biases.reshape(P, 1)

    w2flat = W2.reshape(P * DFF, D)
    out, psum, cnt, sq = pl.pallas_call(
        _moe_kernel,
        grid=(NT, P // EB),
        in_specs=[
            pl.BlockSpec((TB, D), lambda i, p: (i, 0)),
            pl.BlockSpec((P, D), lambda i, p: (0, 0)),
            pl.BlockSpec((P, 1), lambda i, p: (0, 0)),
            pl.BlockSpec((EB, D, DFF), lambda i, p: (p, 0, 0)),
            pl.BlockSpec((EB * DFF, D), lambda i, p: (p, 0)),
        ],
        out_specs=[
            pl.BlockSpec((TB, D), lambda i, p: (i, 0)),
            pl.BlockSpec((1, 1, P), lambda i, p: (i, 0, 0)),
            pl.BlockSpec((1, 1, P), lambda i, p: (i, 0, 0)),
            pl.BlockSpec((1, 1, P), lambda i, p: (i, 0, 0)),
        ],
        out_shape=[
            jax.ShapeDtypeStruct((N, D), jnp.float32),
            jax.ShapeDtypeStruct((NT, 1, P), jnp.float32),
            jax.ShapeDtypeStruct((NT, 1, P), jnp.float32),
            jax.ShapeDtypeStruct((NT, 1, P), jnp.float32),
        ],
        scratch_shapes=[
            pltpu.VMEM((TB, D), jnp.bfloat16),
            pltpu.VMEM((TB, P), jnp.float32),
            pltpu.VMEM((TB, EB * DFF), jnp.bfloat16),
        ],
        compiler_params=pltpu.CompilerParams(
            vmem_limit_bytes=120 * 1024 * 1024,
            dimension_semantics=("parallel", "arbitrary")),
    )(x, keys, bias2d, W1, w2flat)

    mean_prob = jnp.sum(psum, axis=(0, 1)) / N
    load_frac = jnp.sum(cnt, axis=(0, 1)) / (N * K)
    off_bias_loss = OFF_BIAS * P * jnp.sum(mean_prob * load_frac)
    off_var_loss = OFF_VAR * jnp.var(load_frac)
    nudge_loss = NUDGE * jnp.sum(sq[:, 0, 0]) / (N * P)
    loss = off_bias_loss + off_var_loss + nudge_loss
    return out.reshape(B, T, D), loss
